# baseline (speedup 1.0000x reference)
import dataclasses
import functools

import jax
import jax.numpy as jnp
import numpy as np
from jax import lax
from jax.experimental import pallas as pl
from jax.experimental.pallas import tpu as pltpu
from jax.experimental.pallas import tpu_sc as plsc

N = 10000
E = 320000
D = 128
B = 1024
EPS = 1e-5

NC = 2
NS = 16
L = 16

NPAD = 10240
NODES_PER_TILE = NPAD // NS
EPT_H = E // NS
EPT_F = E // (NC * NS)
CE = 2000
CE2 = 2304
NH = EPT_H // CE
NF = EPT_F // CE
GC = 128
CAP = EPT_F + 64 + 2 * GC
ACC_ROWS = 1280
DUMMY = B
BPT = B // NS
RSQRT_MAGIC = np.int32(0x5F3759DF)


def _rsqrt_newton(x):
    y = plsc.bitcast(RSQRT_MAGIC - lax.shift_right_logical(plsc.bitcast(x, jnp.int32), 1), jnp.float32)
    for _ in range(3):
        y = y * (1.5 - 0.5 * x * y * y)
    return y


def _aligned128(off):
    a = off // 128 * 128
    a = jnp.minimum(a, E - CE2)
    return pl.multiple_of(a, 128)

def _sc_kernel_body(edges, agents, state, out, s2h,
                    inv_t, dinv_t, hist_t, histred, srcs, slots,
                    ebuf0, ebuf1, rows0, rows1,
                    agents_t, idxs0, idxs1, idxd0, idxd1, fidx, fwbuf, dloc,
                    sem_a, sem_b, sg0, sg1, ss0, ss1,
                    acc_sh, hist_sh, inv_sh, dinv_sh):
    c = lax.axis_index("c")
    s = lax.axis_index("s")
    wid = c * NS + s
    zeros16 = jnp.zeros((L,), jnp.int32)
    ones16 = jnp.ones((L,), jnp.int32)
    iota16 = lax.iota(jnp.int32, L)
    ebufs = (ebuf0, ebuf1)
    lsems = (sem_a, sem_b)

    hbase = s * EPT_H
    d0 = pltpu.async_copy(edges.at[:, pl.ds(_aligned128(hbase), CE2)], ebuf0, sem_a)

    @pl.loop(0, NPAD // L)
    def _(i):
        hist_t[pl.ds(i * L, L)] = zeros16
    pltpu.sync_copy(agents.at[:], agents_t)

    @pl.when(s == 0)
    def _():
        @pl.loop(0, NPAD // L)
        def _(i):
            inv_t[pl.ds(i * L, L)] = jnp.full((L,), -1, jnp.int32)
        @pl.loop(0, B // L)
        def _(j):
            idx = agents_t[pl.ds(j * L, L)]
            plsc.store_scatter(inv_t, [idx], iota16 + j * L)
        pltpu.sync_copy(inv_t, inv_sh)

    _scope1 = jax.named_scope("ph1_hist"); _scope1.__enter__()
    pend = d0
    for ch in range(NH):
        b = ch % 2
        pend.wait()
        if ch + 1 < NH:
            pend = pltpu.async_copy(
                edges.at[:, pl.ds(_aligned128(hbase + (ch + 1) * CE), CE2)], ebufs[1 - b], lsems[1 - b])
        buf = ebufs[b]
        shift = hbase + ch * CE - _aligned128(hbase + ch * CE)

        @functools.partial(lax.fori_loop, 0, CE // L, init_val=None, unroll=5)
        def _(j, _):
            d16 = buf[1, pl.ds(shift + j * L, L)]
            plsc.addupdate_scatter(hist_t, [d16], ones16)
            return None
    pltpu.sync_copy(hist_t, hist_sh.at[s])

    @pl.loop(0, ACC_ROWS // NS)
    def _(r):
        for q in range(D // L):
            rows0[r, pl.ds(q * L, L)] = jnp.zeros((L,), jnp.float32)
    pltpu.sync_copy(rows0.at[pl.ds(0, ACC_ROWS // NS)],
                    acc_sh.at[pl.ds(s * (ACC_ROWS // NS), ACC_ROWS // NS)])

    plsc.subcore_barrier()
    _scope1.__exit__(None, None, None)

    _scope2 = jax.named_scope("ph2_dinv_prescale"); _scope2.__enter__()
    nbase = s * NODES_PER_TILE
    pltpu.sync_copy(hist_sh.at[:, pl.ds(nbase, NODES_PER_TILE)], histred)
    @pl.loop(0, NODES_PER_TILE // L)
    def _(i):
        acc16 = ones16
        for t in range(NS):
            acc16 = acc16 + histred[t, pl.ds(i * L, L)]
        dloc[pl.ds(i * L, L)] = _rsqrt_newton(acc16.astype(jnp.float32))
    pltpu.sync_copy(dloc, dinv_sh.at[pl.ds(nbase, NODES_PER_TILE)])

    NCH2 = NODES_PER_TILE // 80
    prows = (rows0, rows1)
    plsems = (sg0, sg1)
    pssems = (ss0, ss1)

    def _node0(cc):
        return nbase + cc * 80

    @pl.when(_node0(0) < N)
    def _():
        pltpu.async_copy(state.at[pl.ds(_node0(0), 80)], rows0.at[pl.ds(0, 80)], sg0)
    for cc in range(NCH2):
        b = cc % 2
        buf = prows[b]
        node0 = _node0(cc)
        @pl.when(node0 < N)
        def _():
            pltpu.make_async_copy(state.at[pl.ds(node0, 80)], buf.at[pl.ds(0, 80)], plsems[b]).wait()
        if cc + 1 < NCH2:
            if cc >= 1:
                @pl.when(_node0(cc - 1) < N)
                def _():
                    pltpu.make_async_copy(prows[1 - b].at[pl.ds(0, 80)],
                                          s2h.at[pl.ds(c * N + _node0(cc - 1), 80)],
                                          pssems[1 - b]).wait()
            @pl.when(_node0(cc + 1) < N)
            def _():
                pltpu.async_copy(state.at[pl.ds(_node0(cc + 1), 80)],
                                 prows[1 - b].at[pl.ds(0, 80)], plsems[1 - b])
        @pl.when(node0 < N)
        def _():
            @pl.loop(0, 80 // L)
            def _(g):
                w16 = dloc[pl.ds(cc * 80 + g * L, L)]
                for l in range(L):
                    r = g * L + l
                    w = w16[l]
                    for q in range(D // L):
                        buf[r, pl.ds(q * L, L)] = buf[r, pl.ds(q * L, L)] * w
            pltpu.async_copy(buf.at[pl.ds(0, 80)],
                             s2h.at[pl.ds(c * N + node0, 80)], pssems[b])
    for cc in (NCH2 - 2, NCH2 - 1):
        b = cc % 2
        @pl.when(_node0(cc) < N)
        def _():
            pltpu.make_async_copy(prows[b].at[pl.ds(0, 80)],
                                  s2h.at[pl.ds(c * N + _node0(cc), 80)],
                                  pssems[b]).wait()

    plsc.subcore_barrier()
    _scope2.__exit__(None, None, None)

    _scope3 = jax.named_scope("ph3_filter_accum"); _scope3.__enter__()
    pltpu.sync_copy(inv_sh, inv_t)
    pltpu.sync_copy(dinv_sh, dinv_t)

    fbase = wid * EPT_F
    cN = c * N

    def _process_lists(count):
        for i in range(2 * GC // L):
            srcs[pl.ds(count + i * L, L)] = iota16 + i * L
            slots[pl.ds(count + i * L, L)] = iota16 + (DUMMY + i * L)

        npair = (count + 2 * GC - 1) // (2 * GC)

        def _pair(k, carry):
            @pl.when(k > 0)
            def _():
                pltpu.make_async_copy(rows0, acc_sh.at[idxd0], ss0).wait()
                pltpu.make_async_copy(rows1, acc_sh.at[idxd1], ss1).wait()
            base = k * 2 * GC
            for q in range(GC // L):
                idxs0[pl.ds(q * L, L)] = srcs[pl.ds(base + q * L, L)] + cN
                idxd0[pl.ds(q * L, L)] = slots[pl.ds(base + q * L, L)]
                idxs1[pl.ds(q * L, L)] = srcs[pl.ds(base + GC + q * L, L)] + cN
                idxd1[pl.ds(q * L, L)] = slots[pl.ds(base + GC + q * L, L)]
            g0 = pltpu.async_copy(s2h.at[idxs0], rows0, sg0)
            g1 = pltpu.async_copy(s2h.at[idxs1], rows1, sg1)
            g0.wait()
            pltpu.async_copy(rows0, acc_sh.at[idxd0], ss0, add=True)
            g1.wait()
            pltpu.async_copy(rows1, acc_sh.at[idxd1], ss1, add=True)
            return carry

        lax.fori_loop(0, npair, _pair, jnp.int32(0))

        @pl.when(npair > 0)
        def _():
            pltpu.make_async_copy(rows0, acc_sh.at[idxd0], ss0).wait()
            pltpu.make_async_copy(rows1, acc_sh.at[idxd1], ss1).wait()

    count = jnp.int32(0)
    pende = pltpu.async_copy(edges.at[:, pl.ds(_aligned128(fbase), CE2)], ebuf0, sem_a)
    for ch in range(NF):
        b = ch % 2
        pende.wait()
        if ch + 1 < NF:
            off = fbase + (ch + 1) * CE
            pende = pltpu.async_copy(edges.at[:, pl.ds(_aligned128(off), CE2)], ebufs[1 - b], lsems[1 - b])
        eb = ebufs[b]
        fshift = fbase + ch * CE - _aligned128(fbase + ch * CE)

        def _step(j, cnt):
            d16 = eb[1, pl.ds(fshift + j * L, L)]
            s16 = eb[0, pl.ds(fshift + j * L, L)]
            iv = plsc.load_gather(inv_t, [d16])
            msk = iv >= 0
            plsc.store_compressed(srcs.at[pl.ds(cnt, L)], s16, mask=msk)
            plsc.store_compressed(slots.at[pl.ds(cnt, L)], iv, mask=msk)
            return cnt + plsc.all_reduce_population_count(msk)[0]

        count = lax.fori_loop(0, CE // L, _step, count, unroll=5)

    def _self_step(j, cnt):
        b16 = iota16 + (s * BPT + j * L)
        nodes = agents_t[pl.ds(s * BPT + j * L, L)]
        iv = plsc.load_gather(inv_t, [nodes])
        msk = jnp.logical_and(iv == b16, c == 0)
        plsc.store_compressed(srcs.at[pl.ds(cnt, L)], nodes, mask=msk)
        plsc.store_compressed(slots.at[pl.ds(cnt, L)], b16, mask=msk)
        return cnt + plsc.all_reduce_population_count(msk)[0]

    count = lax.fori_loop(0, BPT // L, _self_step, count)
    _scope3p = jax.named_scope("ph3p_pairs"); _scope3p.__enter__()
    _process_lists(count)
    _scope3p.__exit__(None, None, None)

    plsc.subcore_barrier()
    _scope3.__exit__(None, None, None)

    _scope4 = jax.named_scope("ph4_out"); _scope4.__enter__()
    b0 = s * BPT
    @pl.loop(0, BPT // L)
    def _(j):
        nodes = agents_t[pl.ds(b0 + j * L, L)]
        iv = plsc.load_gather(inv_t, [nodes])
        fidx[pl.ds(j * L, L)] = iv
        fwbuf[pl.ds(j * L, L)] = plsc.load_gather(dinv_t, [nodes])
    pltpu.sync_copy(acc_sh.at[fidx], rows0.at[pl.ds(0, BPT)])
    @pl.loop(0, BPT // L)
    def _(g):
        w16 = fwbuf[pl.ds(g * L, L)]
        for l in range(L):
            r = g * L + l
            w = w16[l]
            for q in range(D // L):
                rows0[r, pl.ds(q * L, L)] = rows0[r, pl.ds(q * L, L)] * w
    pltpu.sync_copy(rows0.at[pl.ds(0, BPT)], out.at[c, pl.ds(b0, BPT)])
    _scope4.__exit__(None, None, None)


@jax.jit
def _sc_gcn(edges, agent_i, state):
    mesh = plsc.VectorSubcoreMesh(core_axis_name="c", subcore_axis_name="s")
    cp = pltpu.CompilerParams()
    if "needs_layout_passes" in pltpu.CompilerParams.__dataclass_fields__:
        cp = dataclasses.replace(cp, needs_layout_passes=False)
    kern = pl.kernel(
        _sc_kernel_body,
        out_type=(jax.ShapeDtypeStruct((NC, B, D), jnp.float32),
                  jax.ShapeDtypeStruct((NC * N, D), jnp.float32)),
        mesh=mesh,
        compiler_params=cp,
        scratch_types=[
            pltpu.VMEM((NPAD,), jnp.int32),
            pltpu.VMEM((NPAD,), jnp.float32),
            pltpu.VMEM((NPAD,), jnp.int32),
            pltpu.VMEM((NS, NODES_PER_TILE), jnp.int32),
            pltpu.VMEM((CAP,), jnp.int32),
            pltpu.VMEM((CAP,), jnp.int32),
            pltpu.VMEM((2, CE2), jnp.int32),
            pltpu.VMEM((2, CE2), jnp.int32),
            pltpu.VMEM((GC, D), jnp.float32),
            pltpu.VMEM((GC, D), jnp.float32),
            pltpu.VMEM((B,), jnp.int32),
            pltpu.VMEM((GC,), jnp.int32),
            pltpu.VMEM((GC,), jnp.int32),
            pltpu.VMEM((GC,), jnp.int32),
            pltpu.VMEM((GC,), jnp.int32),
            pltpu.VMEM((BPT,), jnp.int32),
            pltpu.VMEM((BPT,), jnp.float32),
            pltpu.VMEM((NODES_PER_TILE,), jnp.float32),
            pltpu.SemaphoreType.DMA,
            pltpu.SemaphoreType.DMA,
            pltpu.SemaphoreType.DMA,
            pltpu.SemaphoreType.DMA,
            pltpu.SemaphoreType.DMA,
            pltpu.SemaphoreType.DMA,
            pltpu.VMEM_SHARED((ACC_ROWS, D), jnp.float32),
            pltpu.VMEM_SHARED((NS, NPAD), jnp.int32),
            pltpu.VMEM_SHARED((NPAD,), jnp.int32),
            pltpu.VMEM_SHARED((NPAD,), jnp.float32),
        ],
    )
    partial, _ = kern(edges, agent_i, state)
    return partial


def _ln(x, w, b):
    mu = jnp.mean(x, axis=-1, keepdims=True)
    var = jnp.mean((x - mu) ** 2, axis=-1, keepdims=True)
    return (x - mu) * lax.rsqrt(var + EPS) * w + b


def _tc_body(p_ref, action_ref, Wg_ref, bg_ref, W1_ref, b1_ref, g1_ref,
             be1_ref, W2_ref, b2_ref, g2_ref, be2_ref, Wa_ref, ba_ref,
             Wq_ref, bq_ref, o_ref):
    rows = p_ref[0] + p_ref[1]
    x = jnp.dot(rows, Wg_ref[...], preferred_element_type=jnp.float32, precision=lax.Precision.HIGHEST) + bg_ref[...]
    h = jnp.maximum(x, 0.0)
    sv = jnp.dot(h, W1_ref[...], preferred_element_type=jnp.float32, precision=lax.Precision.HIGHEST) + b1_ref[...]
    sv = _ln(sv, g1_ref[...], be1_ref[...])
    sv = jnp.maximum(sv, 0.0)
    sv = jnp.dot(sv, W2_ref[...], preferred_element_type=jnp.float32, precision=lax.Precision.HIGHEST) + b2_ref[...]
    sv = _ln(sv, g2_ref[...], be2_ref[...])
    av = jnp.dot(action_ref[...], Wa_ref[...], preferred_element_type=jnp.float32, precision=lax.Precision.HIGHEST) + ba_ref[...]
    sav = jnp.maximum(sv + av, 0.0)
    o_ref[...] = jnp.dot(sav, Wq_ref[...], preferred_element_type=jnp.float32, precision=lax.Precision.HIGHEST) + bq_ref[...]


@jax.jit
def _tc_mlp(p, action, Wg, bg, W1, b1, g1, be1, W2, b2, g2, be2, Wa, ba, Wq, bq):
    return pl.pallas_call(
        _tc_body,
        out_shape=jax.ShapeDtypeStruct((B, 1), jnp.float32),
    )(p, action, Wg, bg, W1, b1, g1, be1, W2, b2, g2, be2, Wa, ba, Wq, bq)


def kernel(state, action, edge_index, agent_i, Wg, bg, W1, b1, g1, be1,
           W2, b2, g2, be2, Wa, ba, Wq, bq):
    partial = _sc_gcn(edge_index, agent_i, state)
    return _tc_mlp(partial, action,
                   Wg, bg.reshape(1, -1),
                   W1, b1.reshape(1, -1), g1.reshape(1, -1), be1.reshape(1, -1),
                   W2, b2.reshape(1, -1), g2.reshape(1, -1), be2.reshape(1, -1),
                   Wa, ba.reshape(1, -1),
                   Wq, bq.reshape(1, 1))


# speedup vs baseline: 1.0526x; 1.0526x over previous
import dataclasses
import functools

import jax
import jax.numpy as jnp
import numpy as np
from jax import lax
from jax.experimental import pallas as pl
from jax.experimental.pallas import tpu as pltpu
from jax.experimental.pallas import tpu_sc as plsc

N = 10000
E = 320000
D = 128
B = 1024
EPS = 1e-5

NC = 2
NS = 16
L = 16

NPAD = 10240
NODES_PER_TILE = NPAD // NS
EPT_H = E // NS
EPT_F = E // (NC * NS)
CE = 2000
CE2 = 2304
NH = EPT_H // CE
NF = EPT_F // CE
GC = 128
CAP = EPT_F + 64 + 2 * GC
ACC_ROWS = 1280
DUMMY = B
BPT = B // NS
RSQRT_MAGIC = np.int32(0x5F3759DF)


def _rsqrt_newton(x):
    y = plsc.bitcast(RSQRT_MAGIC - lax.shift_right_logical(plsc.bitcast(x, jnp.int32), 1), jnp.float32)
    for _ in range(3):
        y = y * (1.5 - 0.5 * x * y * y)
    return y


def _aligned128(off):
    a = off // 128 * 128
    a = jnp.minimum(a, E - CE2)
    return pl.multiple_of(a, 128)

def _sc_kernel_body(edges, agents, state, out, s2h,
                    inv_t, dinv_t, hist_t, histred, srcs, slots,
                    ebuf0, ebuf1, rows0, rows1,
                    agents_t, idxs0, idxs1, idxd0, idxd1, fidx, fwbuf, dloc,
                    sem_a, sem_b, sg0, sg1, ss0, ss1,
                    acc_sh, hist_sh, inv_sh, dinv_sh):
    c = lax.axis_index("c")
    s = lax.axis_index("s")
    wid = c * NS + s
    zeros16 = jnp.zeros((L,), jnp.int32)
    ones16 = jnp.ones((L,), jnp.int32)
    iota16 = lax.iota(jnp.int32, L)
    ebufs = (ebuf0, ebuf1)
    lsems = (sem_a, sem_b)

    hbase = s * EPT_H
    d0 = pltpu.async_copy(edges.at[:, pl.ds(_aligned128(hbase), CE2)], ebuf0, sem_a)

    @pl.loop(0, NPAD // L)
    def _(i):
        hist_t[pl.ds(i * L, L)] = zeros16
    pltpu.sync_copy(agents.at[:], agents_t)

    @pl.when(s == 0)
    def _():
        @pl.loop(0, NPAD // L)
        def _(i):
            inv_t[pl.ds(i * L, L)] = jnp.full((L,), -1, jnp.int32)
        @pl.loop(0, B // L)
        def _(j):
            idx = agents_t[pl.ds(j * L, L)]
            plsc.store_scatter(inv_t, [idx], iota16 + j * L)
        pltpu.sync_copy(inv_t, inv_sh)

    _scope1 = jax.named_scope("ph1_hist"); _scope1.__enter__()
    pend = d0
    for ch in range(NH):
        b = ch % 2
        pend.wait()
        if ch + 1 < NH:
            pend = pltpu.async_copy(
                edges.at[:, pl.ds(_aligned128(hbase + (ch + 1) * CE), CE2)], ebufs[1 - b], lsems[1 - b])
        buf = ebufs[b]
        shift = hbase + ch * CE - _aligned128(hbase + ch * CE)

        @functools.partial(lax.fori_loop, 0, CE // L, init_val=None, unroll=5)
        def _(j, _):
            d16 = buf[1, pl.ds(shift + j * L, L)]
            plsc.addupdate_scatter(hist_t, [d16], ones16)
            return None
    pltpu.sync_copy(hist_t, hist_sh.at[s])

    @pl.loop(0, ACC_ROWS // NS)
    def _(r):
        for q in range(D // L):
            rows0[r, pl.ds(q * L, L)] = jnp.zeros((L,), jnp.float32)
    pltpu.sync_copy(rows0.at[pl.ds(0, ACC_ROWS // NS)],
                    acc_sh.at[pl.ds(s * (ACC_ROWS // NS), ACC_ROWS // NS)])

    plsc.subcore_barrier()
    _scope1.__exit__(None, None, None)

    _scope2 = jax.named_scope("ph2_dinv_prescale"); _scope2.__enter__()
    nbase = s * NODES_PER_TILE
    pltpu.sync_copy(hist_sh.at[:, pl.ds(nbase, NODES_PER_TILE)], histred)
    @pl.loop(0, NODES_PER_TILE // L)
    def _(i):
        acc16 = ones16
        for t in range(NS):
            acc16 = acc16 + histred[t, pl.ds(i * L, L)]
        dloc[pl.ds(i * L, L)] = _rsqrt_newton(acc16.astype(jnp.float32))
    pltpu.sync_copy(dloc, dinv_sh.at[pl.ds(nbase, NODES_PER_TILE)])

    NCH2 = NODES_PER_TILE // 80
    prows = (rows0, rows1)
    plsems = (sg0, sg1)
    pssems = (ss0, ss1)

    def _node0(cc):
        return nbase + cc * 80

    @pl.when(_node0(0) < N)
    def _():
        pltpu.async_copy(state.at[pl.ds(_node0(0), 80)], rows0.at[pl.ds(0, 80)], sg0)
    for cc in range(NCH2):
        b = cc % 2
        buf = prows[b]
        node0 = _node0(cc)
        @pl.when(node0 < N)
        def _():
            pltpu.make_async_copy(state.at[pl.ds(node0, 80)], buf.at[pl.ds(0, 80)], plsems[b]).wait()
        if cc + 1 < NCH2:
            if cc >= 1:
                @pl.when(_node0(cc - 1) < N)
                def _():
                    pltpu.make_async_copy(prows[1 - b].at[pl.ds(0, 80)],
                                          s2h.at[pl.ds(c * N + _node0(cc - 1), 80)],
                                          pssems[1 - b]).wait()
            @pl.when(_node0(cc + 1) < N)
            def _():
                pltpu.async_copy(state.at[pl.ds(_node0(cc + 1), 80)],
                                 prows[1 - b].at[pl.ds(0, 80)], plsems[1 - b])
        @pl.when(node0 < N)
        def _():
            @pl.loop(0, 80 // L)
            def _(g):
                w16 = dloc[pl.ds(cc * 80 + g * L, L)]
                for l in range(L):
                    r = g * L + l
                    w = w16[l]
                    for q in range(D // L):
                        buf[r, pl.ds(q * L, L)] = buf[r, pl.ds(q * L, L)] * w
            pltpu.async_copy(buf.at[pl.ds(0, 80)],
                             s2h.at[pl.ds(c * N + node0, 80)], pssems[b])
    for cc in (NCH2 - 2, NCH2 - 1):
        b = cc % 2
        @pl.when(_node0(cc) < N)
        def _():
            pltpu.make_async_copy(prows[b].at[pl.ds(0, 80)],
                                  s2h.at[pl.ds(c * N + _node0(cc), 80)],
                                  pssems[b]).wait()

    plsc.subcore_barrier()
    _scope2.__exit__(None, None, None)

    _scope3 = jax.named_scope("ph3_filter_accum"); _scope3.__enter__()
    pltpu.sync_copy(inv_sh, inv_t)
    pltpu.sync_copy(dinv_sh, dinv_t)

    fbase = wid * EPT_F
    cN = c * N

    def _process_lists(count):
        for i in range(2 * GC // L):
            srcs[pl.ds(count + i * L, L)] = iota16 + i * L
            slots[pl.ds(count + i * L, L)] = iota16 + (DUMMY + i * L)

        npair = (count + 2 * GC - 1) // (2 * GC)

        def _pair(k, carry):
            @pl.when(k > 0)
            def _():
                pltpu.make_async_copy(rows0, acc_sh.at[idxd0], ss0).wait()
                pltpu.make_async_copy(rows1, acc_sh.at[idxd1], ss1).wait()
            base = k * 2 * GC
            for q in range(GC // L):
                idxs0[pl.ds(q * L, L)] = srcs[pl.ds(base + q * L, L)] + cN
                idxd0[pl.ds(q * L, L)] = slots[pl.ds(base + q * L, L)]
                idxs1[pl.ds(q * L, L)] = srcs[pl.ds(base + GC + q * L, L)] + cN
                idxd1[pl.ds(q * L, L)] = slots[pl.ds(base + GC + q * L, L)]
            g0 = pltpu.async_copy(s2h.at[idxs0], rows0, sg0)
            g1 = pltpu.async_copy(s2h.at[idxs1], rows1, sg1)
            g0.wait()
            pltpu.async_copy(rows0, acc_sh.at[idxd0], ss0, add=True)
            g1.wait()
            pltpu.async_copy(rows1, acc_sh.at[idxd1], ss1, add=True)
            return carry

        lax.fori_loop(0, npair, _pair, jnp.int32(0))

        @pl.when(npair > 0)
        def _():
            pltpu.make_async_copy(rows0, acc_sh.at[idxd0], ss0).wait()
            pltpu.make_async_copy(rows1, acc_sh.at[idxd1], ss1).wait()

    count = jnp.int32(0)
    pende = pltpu.async_copy(edges.at[:, pl.ds(_aligned128(fbase), CE2)], ebuf0, sem_a)
    for ch in range(NF):
        b = ch % 2
        pende.wait()
        if ch + 1 < NF:
            off = fbase + (ch + 1) * CE
            pende = pltpu.async_copy(edges.at[:, pl.ds(_aligned128(off), CE2)], ebufs[1 - b], lsems[1 - b])
        eb = ebufs[b]
        fshift = fbase + ch * CE - _aligned128(fbase + ch * CE)

        def _step(j, cnt):
            d16 = eb[1, pl.ds(fshift + j * L, L)]
            s16 = eb[0, pl.ds(fshift + j * L, L)]
            iv = plsc.load_gather(inv_t, [d16])
            msk = iv >= 0
            plsc.store_compressed(srcs.at[pl.ds(cnt, L)], s16, mask=msk)
            plsc.store_compressed(slots.at[pl.ds(cnt, L)], iv, mask=msk)
            return cnt + plsc.all_reduce_population_count(msk)[0]

        count = lax.fori_loop(0, CE // L, _step, count, unroll=5)

    def _self_step(j, cnt):
        b16 = iota16 + (s * BPT + j * L)
        nodes = agents_t[pl.ds(s * BPT + j * L, L)]
        iv = plsc.load_gather(inv_t, [nodes])
        msk = jnp.logical_and(iv == b16, c == 0)
        plsc.store_compressed(srcs.at[pl.ds(cnt, L)], nodes, mask=msk)
        plsc.store_compressed(slots.at[pl.ds(cnt, L)], b16, mask=msk)
        return cnt + plsc.all_reduce_population_count(msk)[0]

    count = lax.fori_loop(0, BPT // L, _self_step, count)
    _scope3p = jax.named_scope("ph3p_pairs"); _scope3p.__enter__()
    _process_lists(count)
    _scope3p.__exit__(None, None, None)

    plsc.subcore_barrier()
    _scope3.__exit__(None, None, None)

    _scope4 = jax.named_scope("ph4_out"); _scope4.__enter__()
    b0 = s * BPT
    @pl.loop(0, BPT // L)
    def _(j):
        nodes = agents_t[pl.ds(b0 + j * L, L)]
        iv = plsc.load_gather(inv_t, [nodes])
        fidx[pl.ds(j * L, L)] = iv
        fwbuf[pl.ds(j * L, L)] = plsc.load_gather(dinv_t, [nodes])
    pltpu.sync_copy(acc_sh.at[fidx], rows0.at[pl.ds(0, BPT)])
    @pl.loop(0, BPT // L)
    def _(g):
        w16 = fwbuf[pl.ds(g * L, L)]
        for l in range(L):
            r = g * L + l
            w = w16[l]
            for q in range(D // L):
                rows0[r, pl.ds(q * L, L)] = rows0[r, pl.ds(q * L, L)] * w
    pltpu.sync_copy(rows0.at[pl.ds(0, BPT)], out.at[c, pl.ds(b0, BPT)])
    _scope4.__exit__(None, None, None)


@jax.jit
def _sc_gcn(edges, agent_i, state):
    mesh = plsc.VectorSubcoreMesh(core_axis_name="c", subcore_axis_name="s")
    cp = pltpu.CompilerParams()
    if "needs_layout_passes" in pltpu.CompilerParams.__dataclass_fields__:
        cp = dataclasses.replace(cp, needs_layout_passes=False)
    kern = pl.kernel(
        _sc_kernel_body,
        out_type=(jax.ShapeDtypeStruct((NC, B, D), jnp.float32),
                  jax.ShapeDtypeStruct((NC * N, D), jnp.float32)),
        mesh=mesh,
        compiler_params=cp,
        scratch_types=[
            pltpu.VMEM((NPAD,), jnp.int32),
            pltpu.VMEM((NPAD,), jnp.float32),
            pltpu.VMEM((NPAD,), jnp.int32),
            pltpu.VMEM((NS, NODES_PER_TILE), jnp.int32),
            pltpu.VMEM((CAP,), jnp.int32),
            pltpu.VMEM((CAP,), jnp.int32),
            pltpu.VMEM((2, CE2), jnp.int32),
            pltpu.VMEM((2, CE2), jnp.int32),
            pltpu.VMEM((GC, D), jnp.float32),
            pltpu.VMEM((GC, D), jnp.float32),
            pltpu.VMEM((B,), jnp.int32),
            pltpu.VMEM((GC,), jnp.int32),
            pltpu.VMEM((GC,), jnp.int32),
            pltpu.VMEM((GC,), jnp.int32),
            pltpu.VMEM((GC,), jnp.int32),
            pltpu.VMEM((BPT,), jnp.int32),
            pltpu.VMEM((BPT,), jnp.float32),
            pltpu.VMEM((NODES_PER_TILE,), jnp.float32),
            pltpu.SemaphoreType.DMA,
            pltpu.SemaphoreType.DMA,
            pltpu.SemaphoreType.DMA,
            pltpu.SemaphoreType.DMA,
            pltpu.SemaphoreType.DMA,
            pltpu.SemaphoreType.DMA,
            pltpu.VMEM_SHARED((ACC_ROWS, D), jnp.float32),
            pltpu.VMEM_SHARED((NS, NPAD), jnp.int32),
            pltpu.VMEM_SHARED((NPAD,), jnp.int32),
            pltpu.VMEM_SHARED((NPAD,), jnp.float32),
        ],
    )
    partial, _ = kern(edges, agent_i, state)
    return partial


def _ln(x, w, b):
    mu = jnp.mean(x, axis=-1, keepdims=True)
    var = jnp.mean((x - mu) ** 2, axis=-1, keepdims=True)
    return (x - mu) * lax.rsqrt(var + EPS) * w + b


def _tc_body(p_ref, action_ref, Wg_ref, bg_ref, W1_ref, b1_ref, g1_ref,
             be1_ref, W2_ref, b2_ref, g2_ref, be2_ref, Wa_ref, ba_ref,
             Wq_ref, bq_ref, o_ref):
    rows = p_ref[0] + p_ref[1]
    x = jnp.dot(rows, Wg_ref[...], preferred_element_type=jnp.float32) + bg_ref[...]
    h = jnp.maximum(x, 0.0)
    sv = jnp.dot(h, W1_ref[...], preferred_element_type=jnp.float32) + b1_ref[...]
    sv = _ln(sv, g1_ref[...], be1_ref[...])
    sv = jnp.maximum(sv, 0.0)
    sv = jnp.dot(sv, W2_ref[...], preferred_element_type=jnp.float32) + b2_ref[...]
    sv = _ln(sv, g2_ref[...], be2_ref[...])
    av = jnp.dot(action_ref[...], Wa_ref[...], preferred_element_type=jnp.float32) + ba_ref[...]
    sav = jnp.maximum(sv + av, 0.0)
    o_ref[...] = jnp.dot(sav, Wq_ref[...], preferred_element_type=jnp.float32) + bq_ref[...]


@jax.jit
def _tc_mlp(p, action, Wg, bg, W1, b1, g1, be1, W2, b2, g2, be2, Wa, ba, Wq, bq):
    return pl.pallas_call(
        _tc_body,
        out_shape=jax.ShapeDtypeStruct((B, 1), jnp.float32),
    )(p, action, Wg, bg, W1, b1, g1, be1, W2, b2, g2, be2, Wa, ba, Wq, bq)


def kernel(state, action, edge_index, agent_i, Wg, bg, W1, b1, g1, be1,
           W2, b2, g2, be2, Wa, ba, Wq, bq):
    partial = _sc_gcn(edge_index, agent_i, state)
    return _tc_mlp(partial, action,
                   Wg, bg.reshape(1, -1),
                   W1, b1.reshape(1, -1), g1.reshape(1, -1), be1.reshape(1, -1),
                   W2, b2.reshape(1, -1), g2.reshape(1, -1), be2.reshape(1, -1),
                   Wa, ba.reshape(1, -1),
                   Wq, bq.reshape(1, 1))


# speedup vs baseline: 1.0578x; 1.0050x over previous
import dataclasses
import functools

import jax
import jax.numpy as jnp
import numpy as np
from jax import lax
from jax.experimental import pallas as pl
from jax.experimental.pallas import tpu as pltpu
from jax.experimental.pallas import tpu_sc as plsc

N = 10000
E = 320000
D = 128
B = 1024
EPS = 1e-5

NC = 2
NS = 16
L = 16

NPAD = 10240
NODES_PER_TILE = NPAD // NS
EPT_H = E // NS
EPT_F = E // (NC * NS)
CE = 2000
CE2 = 2304
NH = EPT_H // CE
NF = EPT_F // CE
GC = 64
CAP = EPT_F + 64 + 4 * GC
ACC_ROWS = 1280
DUMMY = B
BPT = B // NS
RSQRT_MAGIC = np.int32(0x5F3759DF)


def _rsqrt_newton(x):
    y = plsc.bitcast(RSQRT_MAGIC - lax.shift_right_logical(plsc.bitcast(x, jnp.int32), 1), jnp.float32)
    for _ in range(3):
        y = y * (1.5 - 0.5 * x * y * y)
    return y


def _aligned128(off):
    a = off // 128 * 128
    a = jnp.minimum(a, E - CE2)
    return pl.multiple_of(a, 128)

def _sc_kernel_body(edges, agents, state, out, s2h,
                    inv_t, dinv_t, hist_t, histred, srcs, slots,
                    ebuf0, ebuf1, rows0, rows1, rows2, rows3,
                    agents_t, idxs0, idxs1, idxs2, idxs3,
                    idxd0, idxd1, idxd2, idxd3, fidx, fwbuf, dloc,
                    sem_a, sem_b, sg0, sg1, ss0, ss1,
                    acc_sh, hist_sh, inv_sh, dinv_sh):
    c = lax.axis_index("c")
    s = lax.axis_index("s")
    wid = c * NS + s
    zeros16 = jnp.zeros((L,), jnp.int32)
    ones16 = jnp.ones((L,), jnp.int32)
    iota16 = lax.iota(jnp.int32, L)
    ebufs = (ebuf0, ebuf1)
    lsems = (sem_a, sem_b)

    hbase = s * EPT_H
    d0 = pltpu.async_copy(edges.at[:, pl.ds(_aligned128(hbase), CE2)], ebuf0, sem_a)

    @pl.loop(0, NPAD // L)
    def _(i):
        hist_t[pl.ds(i * L, L)] = zeros16
    pltpu.sync_copy(agents.at[:], agents_t)

    @pl.when(s == 0)
    def _():
        @pl.loop(0, NPAD // L)
        def _(i):
            inv_t[pl.ds(i * L, L)] = jnp.full((L,), -1, jnp.int32)
        @pl.loop(0, B // L)
        def _(j):
            idx = agents_t[pl.ds(j * L, L)]
            plsc.store_scatter(inv_t, [idx], iota16 + j * L)
        pltpu.sync_copy(inv_t, inv_sh)

    _scope1 = jax.named_scope("ph1_hist"); _scope1.__enter__()
    pend = d0
    for ch in range(NH):
        b = ch % 2
        pend.wait()
        if ch + 1 < NH:
            pend = pltpu.async_copy(
                edges.at[:, pl.ds(_aligned128(hbase + (ch + 1) * CE), CE2)], ebufs[1 - b], lsems[1 - b])
        buf = ebufs[b]
        shift = hbase + ch * CE - _aligned128(hbase + ch * CE)

        @functools.partial(lax.fori_loop, 0, CE // L, init_val=None, unroll=5)
        def _(j, _):
            d16 = buf[1, pl.ds(shift + j * L, L)]
            plsc.addupdate_scatter(hist_t, [d16], ones16)
            return None
    pltpu.sync_copy(hist_t, hist_sh.at[s])

    @pl.loop(0, 40)
    def _(r):
        for q in range(D // L):
            rows0[r, pl.ds(q * L, L)] = jnp.zeros((L,), jnp.float32)
    pltpu.sync_copy(rows0.at[pl.ds(0, 40)], acc_sh.at[pl.ds(s * 80, 40)])
    pltpu.sync_copy(rows0.at[pl.ds(0, 40)], acc_sh.at[pl.ds(s * 80 + 40, 40)])

    plsc.subcore_barrier()
    _scope1.__exit__(None, None, None)

    _scope2 = jax.named_scope("ph2_dinv_prescale"); _scope2.__enter__()
    nbase = s * NODES_PER_TILE
    pltpu.sync_copy(hist_sh.at[:, pl.ds(nbase, NODES_PER_TILE)], histred)
    @pl.loop(0, NODES_PER_TILE // L)
    def _(i):
        acc16 = ones16
        for t in range(NS):
            acc16 = acc16 + histred[t, pl.ds(i * L, L)]
        dloc[pl.ds(i * L, L)] = _rsqrt_newton(acc16.astype(jnp.float32))
    pltpu.sync_copy(dloc, dinv_sh.at[pl.ds(nbase, NODES_PER_TILE)])

    CPS = 64
    NCH2 = NODES_PER_TILE // CPS
    prows = (rows0, rows1)
    plsems = (sg0, sg1)
    pssems = (ss0, ss1)

    def _node0(cc):
        return nbase + cc * CPS

    @pl.when(_node0(0) + CPS <= N)
    def _():
        pltpu.async_copy(state.at[pl.ds(_node0(0), CPS)], rows0.at[pl.ds(0, CPS)], sg0)
    for cc in range(NCH2):
        b = cc % 2
        buf = prows[b]
        node0 = _node0(cc)
        @pl.when(node0 + CPS <= N)
        def _():
            pltpu.make_async_copy(state.at[pl.ds(node0, CPS)], buf.at[pl.ds(0, CPS)], plsems[b]).wait()
        if cc + 1 < NCH2:
            if cc >= 1:
                @pl.when(_node0(cc - 1) + CPS <= N)
                def _():
                    pltpu.make_async_copy(prows[1 - b].at[pl.ds(0, CPS)],
                                          s2h.at[pl.ds(c * N + _node0(cc - 1), CPS)],
                                          pssems[1 - b]).wait()
            @pl.when(_node0(cc + 1) + CPS <= N)
            def _():
                pltpu.async_copy(state.at[pl.ds(_node0(cc + 1), CPS)],
                                 prows[1 - b].at[pl.ds(0, CPS)], plsems[1 - b])
        @pl.when(node0 + CPS <= N)
        def _():
            @pl.loop(0, CPS // L)
            def _(g):
                w16 = dloc[pl.ds(cc * CPS + g * L, L)]
                for l in range(L):
                    r = g * L + l
                    w = w16[l]
                    for q in range(D // L):
                        buf[r, pl.ds(q * L, L)] = buf[r, pl.ds(q * L, L)] * w
            pltpu.async_copy(buf.at[pl.ds(0, CPS)],
                             s2h.at[pl.ds(c * N + node0, CPS)], pssems[b])
    for cc in (NCH2 - 2, NCH2 - 1):
        b = cc % 2
        @pl.when(_node0(cc) + CPS <= N)
        def _():
            pltpu.make_async_copy(prows[b].at[pl.ds(0, CPS)],
                                  s2h.at[pl.ds(c * N + _node0(cc), CPS)],
                                  pssems[b]).wait()

    nvalid = jnp.maximum(jnp.minimum(nbase + NODES_PER_TILE, N) - nbase, 0)
    full_end = nbase + nvalid // CPS * CPS

    @pl.when(nvalid % CPS != 0)
    def _():
        pltpu.sync_copy(state.at[pl.ds(full_end, L)], rows0.at[pl.ds(0, L)])
        w16 = dloc[pl.ds(full_end - nbase, L)]
        for l in range(L):
            w = w16[l]
            for q in range(D // L):
                rows0[l, pl.ds(q * L, L)] = rows0[l, pl.ds(q * L, L)] * w
        pltpu.sync_copy(rows0.at[pl.ds(0, L)], s2h.at[pl.ds(c * N + full_end, L)])

    plsc.subcore_barrier()
    _scope2.__exit__(None, None, None)

    _scope3 = jax.named_scope("ph3_filter_accum"); _scope3.__enter__()
    pltpu.sync_copy(inv_sh, inv_t)
    pltpu.sync_copy(dinv_sh, dinv_t)

    fbase = wid * EPT_F
    cN = c * N

    def _process_lists(count):
        for i in range(4 * GC // L):
            srcs[pl.ds(count + i * L, L)] = iota16 + i * L
            slots[pl.ds(count + i * L, L)] = iota16 + (DUMMY + i * L)

        nstep = (count + 4 * GC - 1) // (4 * GC)
        rset = ((rows0, rows1), (rows2, rows3))
        iset = ((idxs0, idxs1), (idxs2, idxs3))
        dset = ((idxd0, idxd1), (idxd2, idxd3))
        gsem = (sg0, sg1)
        ssem = (ss0, ss1)

        def _step(k, carry):
            base = k * 4 * GC
            gd = [None, None, None, None]
            for h in range(2):
                @pl.when(k > 0)
                def _():
                    pltpu.make_async_copy(rset[h][0], acc_sh.at[dset[h][0]], ssem[h]).wait()
                    pltpu.make_async_copy(rset[h][1], acc_sh.at[dset[h][1]], ssem[h]).wait()
                for j in range(2):
                    cb = base + (2 * h + j) * GC
                    for q in range(GC // L):
                        iset[h][j][pl.ds(q * L, L)] = srcs[pl.ds(cb + q * L, L)] + cN
                        dset[h][j][pl.ds(q * L, L)] = slots[pl.ds(cb + q * L, L)]
                    gd[2 * h + j] = pltpu.async_copy(s2h.at[iset[h][j]], rset[h][j], gsem[h])
            for h in range(2):
                for j in range(2):
                    gd[2 * h + j].wait()
                    pltpu.async_copy(rset[h][j], acc_sh.at[dset[h][j]], ssem[h], add=True)
            return carry

        lax.fori_loop(0, nstep, _step, jnp.int32(0))

        @pl.when(nstep > 0)
        def _():
            for h in range(2):
                pltpu.make_async_copy(rset[h][0], acc_sh.at[dset[h][0]], ssem[h]).wait()
                pltpu.make_async_copy(rset[h][1], acc_sh.at[dset[h][1]], ssem[h]).wait()

    count = jnp.int32(0)
    pende = pltpu.async_copy(edges.at[:, pl.ds(_aligned128(fbase), CE2)], ebuf0, sem_a)
    for ch in range(NF):
        b = ch % 2
        pende.wait()
        if ch + 1 < NF:
            off = fbase + (ch + 1) * CE
            pende = pltpu.async_copy(edges.at[:, pl.ds(_aligned128(off), CE2)], ebufs[1 - b], lsems[1 - b])
        eb = ebufs[b]
        fshift = fbase + ch * CE - _aligned128(fbase + ch * CE)

        def _step(j, cnt):
            d16 = eb[1, pl.ds(fshift + j * L, L)]
            s16 = eb[0, pl.ds(fshift + j * L, L)]
            iv = plsc.load_gather(inv_t, [d16])
            msk = iv >= 0
            plsc.store_compressed(srcs.at[pl.ds(cnt, L)], s16, mask=msk)
            plsc.store_compressed(slots.at[pl.ds(cnt, L)], iv, mask=msk)
            return cnt + plsc.all_reduce_population_count(msk)[0]

        count = lax.fori_loop(0, CE // L, _step, count, unroll=5)

    def _self_step(j, cnt):
        b16 = iota16 + (s * BPT + j * L)
        nodes = agents_t[pl.ds(s * BPT + j * L, L)]
        iv = plsc.load_gather(inv_t, [nodes])
        msk = jnp.logical_and(iv == b16, c == 0)
        plsc.store_compressed(srcs.at[pl.ds(cnt, L)], nodes, mask=msk)
        plsc.store_compressed(slots.at[pl.ds(cnt, L)], b16, mask=msk)
        return cnt + plsc.all_reduce_population_count(msk)[0]

    count = lax.fori_loop(0, BPT // L, _self_step, count)
    _scope3p = jax.named_scope("ph3p_pairs"); _scope3p.__enter__()
    _process_lists(count)
    _scope3p.__exit__(None, None, None)

    plsc.subcore_barrier()
    _scope3.__exit__(None, None, None)

    _scope4 = jax.named_scope("ph4_out"); _scope4.__enter__()
    b0 = s * BPT
    @pl.loop(0, BPT // L)
    def _(j):
        nodes = agents_t[pl.ds(b0 + j * L, L)]
        iv = plsc.load_gather(inv_t, [nodes])
        fidx[pl.ds(j * L, L)] = iv
        fwbuf[pl.ds(j * L, L)] = plsc.load_gather(dinv_t, [nodes])
    pltpu.sync_copy(acc_sh.at[fidx], rows0.at[pl.ds(0, BPT)])
    @pl.loop(0, BPT // L)
    def _(g):
        w16 = fwbuf[pl.ds(g * L, L)]
        for l in range(L):
            r = g * L + l
            w = w16[l]
            for q in range(D // L):
                rows0[r, pl.ds(q * L, L)] = rows0[r, pl.ds(q * L, L)] * w
    pltpu.sync_copy(rows0.at[pl.ds(0, BPT)], out.at[c, pl.ds(b0, BPT)])
    _scope4.__exit__(None, None, None)


@jax.jit
def _sc_gcn(edges, agent_i, state):
    mesh = plsc.VectorSubcoreMesh(core_axis_name="c", subcore_axis_name="s")
    cp = pltpu.CompilerParams()
    if "needs_layout_passes" in pltpu.CompilerParams.__dataclass_fields__:
        cp = dataclasses.replace(cp, needs_layout_passes=False)
    kern = pl.kernel(
        _sc_kernel_body,
        out_type=(jax.ShapeDtypeStruct((NC, B, D), jnp.float32),
                  jax.ShapeDtypeStruct((NC * N, D), jnp.float32)),
        mesh=mesh,
        compiler_params=cp,
        scratch_types=[
            pltpu.VMEM((NPAD,), jnp.int32),
            pltpu.VMEM((NPAD,), jnp.float32),
            pltpu.VMEM((NPAD,), jnp.int32),
            pltpu.VMEM((NS, NODES_PER_TILE), jnp.int32),
            pltpu.VMEM((CAP,), jnp.int32),
            pltpu.VMEM((CAP,), jnp.int32),
            pltpu.VMEM((2, CE2), jnp.int32),
            pltpu.VMEM((2, CE2), jnp.int32),
            pltpu.VMEM((GC, D), jnp.float32),
            pltpu.VMEM((GC, D), jnp.float32),
            pltpu.VMEM((GC, D), jnp.float32),
            pltpu.VMEM((GC, D), jnp.float32),
            pltpu.VMEM((B,), jnp.int32),
            pltpu.VMEM((GC,), jnp.int32),
            pltpu.VMEM((GC,), jnp.int32),
            pltpu.VMEM((GC,), jnp.int32),
            pltpu.VMEM((GC,), jnp.int32),
            pltpu.VMEM((GC,), jnp.int32),
            pltpu.VMEM((GC,), jnp.int32),
            pltpu.VMEM((GC,), jnp.int32),
            pltpu.VMEM((GC,), jnp.int32),
            pltpu.VMEM((BPT,), jnp.int32),
            pltpu.VMEM((BPT,), jnp.float32),
            pltpu.VMEM((NODES_PER_TILE,), jnp.float32),
            pltpu.SemaphoreType.DMA,
            pltpu.SemaphoreType.DMA,
            pltpu.SemaphoreType.DMA,
            pltpu.SemaphoreType.DMA,
            pltpu.SemaphoreType.DMA,
            pltpu.SemaphoreType.DMA,
            pltpu.VMEM_SHARED((ACC_ROWS, D), jnp.float32),
            pltpu.VMEM_SHARED((NS, NPAD), jnp.int32),
            pltpu.VMEM_SHARED((NPAD,), jnp.int32),
            pltpu.VMEM_SHARED((NPAD,), jnp.float32),
        ],
    )
    partial, _ = kern(edges, agent_i, state)
    return partial


def _ln(x, w, b):
    mu = jnp.mean(x, axis=-1, keepdims=True)
    var = jnp.mean((x - mu) ** 2, axis=-1, keepdims=True)
    return (x - mu) * lax.rsqrt(var + EPS) * w + b


def _tc_body(p_ref, action_ref, Wg_ref, bg_ref, W1_ref, b1_ref, g1_ref,
             be1_ref, W2_ref, b2_ref, g2_ref, be2_ref, Wa_ref, ba_ref,
             Wq_ref, bq_ref, o_ref):
    rows = p_ref[0] + p_ref[1]
    x = jnp.dot(rows, Wg_ref[...], preferred_element_type=jnp.float32) + bg_ref[...]
    h = jnp.maximum(x, 0.0)
    sv = jnp.dot(h, W1_ref[...], preferred_element_type=jnp.float32) + b1_ref[...]
    sv = _ln(sv, g1_ref[...], be1_ref[...])
    sv = jnp.maximum(sv, 0.0)
    sv = jnp.dot(sv, W2_ref[...], preferred_element_type=jnp.float32) + b2_ref[...]
    sv = _ln(sv, g2_ref[...], be2_ref[...])
    av = jnp.dot(action_ref[...], Wa_ref[...], preferred_element_type=jnp.float32) + ba_ref[...]
    sav = jnp.maximum(sv + av, 0.0)
    o_ref[...] = jnp.dot(sav, Wq_ref[...], preferred_element_type=jnp.float32) + bq_ref[...]


@jax.jit
def _tc_mlp(p, action, Wg, bg, W1, b1, g1, be1, W2, b2, g2, be2, Wa, ba, Wq, bq):
    return pl.pallas_call(
        _tc_body,
        out_shape=jax.ShapeDtypeStruct((B, 1), jnp.float32),
    )(p, action, Wg, bg, W1, b1, g1, be1, W2, b2, g2, be2, Wa, ba, Wq, bq)


def kernel(state, action, edge_index, agent_i, Wg, bg, W1, b1, g1, be1,
           W2, b2, g2, be2, Wa, ba, Wq, bq):
    partial = _sc_gcn(edge_index, agent_i, state)
    return _tc_mlp(partial, action,
                   Wg, bg.reshape(1, -1),
                   W1, b1.reshape(1, -1), g1.reshape(1, -1), be1.reshape(1, -1),
                   W2, b2.reshape(1, -1), g2.reshape(1, -1), be2.reshape(1, -1),
                   Wa, ba.reshape(1, -1),
                   Wq, bq.reshape(1, 1))
